# baseline (speedup 1.0000x reference)
import functools

import jax
import jax.numpy as jnp
from jax import lax
from jax.experimental import pallas as pl
from jax.experimental.pallas import tpu as pltpu
from jax.experimental.pallas import tpu_sc as plsc

N_USERS = 4096
N_NEWS = 10000
N_ENTITY = 30000
N_REL = 40
D = 128
NEIGH = 20

C = D // 16
BATCH = 16
IDX_PER_BATCH = BATCH * NEIGH
IDX_ROWS = 5
IDX_COLS = IDX_PER_BATCH // IDX_ROWS
NWORKERS = 32

NEWS_PAD = 10240
ENT_PAD = 30720

USER_BLK = 256


_GDN = lax.GatherDimensionNumbers(
    offset_dims=(), collapsed_slice_dims=(0,), start_index_map=(0,))


def _gather16(v, idx):
    return lax.gather(v, idx[:, None], dimension_numbers=_GDN,
                      slice_sizes=(1,),
                      mode=lax.GatherScatterMode.PROMISE_IN_BOUNDS)


_LANE = None


def _bsum(v, lane):
    for s in (1, 2, 4, 8):
        v = v + _gather16(v, lax.bitwise_xor(lane, s))
    return v


def _bmax(v, lane):
    for s in (1, 2, 4, 8):
        v = jnp.maximum(v, _gather16(v, lax.bitwise_xor(lane, s)))
    return v


def _bf16_round(x):
    i = lax.bitcast_convert_type(x, jnp.int32)
    r = i + 0x7FFF + lax.bitwise_and(lax.shift_right_logical(i, 16), 1)
    return lax.bitcast_convert_type(lax.bitwise_and(r, -65536), jnp.float32)


def _sqrt16(x):
    xs = jnp.maximum(x, 1e-30)
    i = lax.bitcast_convert_type(xs, jnp.int32)
    y = lax.bitcast_convert_type(0x5F3759DF - lax.shift_right_arithmetic(i, 1),
                                 jnp.float32)
    for _ in range(3):
        y = y * (1.5 - 0.5 * xs * y * y)
    return xs * y


def _make_agg(n_pad):
    cpw = n_pad // NWORKERS
    nb = cpw // BATCH
    mesh = plsc.VectorSubcoreMesh(core_axis_name="c", subcore_axis_name="s")

    @functools.partial(
        pl.kernel,
        out_type=jax.ShapeDtypeStruct((n_pad, D), jnp.float32),
        mesh=mesh,
        scratch_types=[
            pltpu.VMEM((IDX_ROWS, IDX_COLS), jnp.int32),
            pltpu.VMEM((IDX_ROWS, IDX_COLS), jnp.int32),
            pltpu.VMEM((BATCH, D), jnp.float32),
            pltpu.VMEM((IDX_PER_BATCH, D), jnp.float32),
            pltpu.VMEM((IDX_PER_BATCH, D), jnp.float32),
            pltpu.VMEM((48,), jnp.float32),
            pltpu.VMEM((BATCH, D), jnp.float32),
            pltpu.SemaphoreType.DMA,
        ],
    )
    def agg(head_hbm, idx_hbm, relid_hbm, ent_hbm, relemb_hbm, out_hbm,
            idx_v, relid_v, h_v, t_v, r_v, sc_v, out_v, sem):
        wid = lax.axis_index("s") * 2 + lax.axis_index("c")
        lane = lax.broadcasted_iota(jnp.int32, (16,), 0)
        zero16 = jnp.zeros((16,), jnp.float32)
        lane0 = jnp.zeros((16,), jnp.int32)

        def batch_body(b, _):
            gb = wid * nb + b
            s = gb * BATCH

            pltpu.sync_copy(idx_hbm.at[gb], idx_v)
            pltpu.sync_copy(relid_hbm.at[gb], relid_v)
            pltpu.sync_copy(head_hbm.at[pl.ds(s, BATCH)], h_v)
            copies = [
                pltpu.async_copy(ent_hbm.at[idx_v.at[j]],
                                 t_v.at[pl.ds(j * IDX_COLS, IDX_COLS)], sem)
                for j in range(IDX_ROWS)
            ] + [
                pltpu.async_copy(relemb_hbm.at[relid_v.at[j]],
                                 r_v.at[pl.ds(j * IDX_COLS, IDX_COLS)], sem)
                for j in range(IDX_ROWS)
            ]
            for cp in copies:
                cp.wait()

            def row_body(i, _):
                f0 = i * NEIGH
                h_c = [h_v[i, pl.ds(c * 16, 16)] for c in range(C)]

                def k_s2(k, s2):
                    f = f0 + k
                    new = []
                    for c in range(C):
                        rl = r_v[f, pl.ds(c * 16, 16)]
                        t = t_v[f, pl.ds(c * 16, 16)]
                        new.append(s2[c] + rl * rl)
                        r_v[f, pl.ds(c * 16, 16)] = _bf16_round(t * rl)
                    return tuple(new)

                s2 = lax.fori_loop(0, NEIGH, k_s2,
                                   tuple(zero16 for _ in range(C)))
                w_c = [_bf16_round(jnp.abs(h_c[c]) * _sqrt16(s2[c]))
                       for c in range(C)]

                def k_score(k, carry):
                    p1, p2 = carry
                    f = f0 + k
                    acc = zero16
                    for c in range(C):
                        acc = acc + r_v[f, pl.ds(c * 16, 16)] * w_c[c]
                    sc = _bsum(acc, lane)
                    sc = sc * sc
                    p1 = jnp.where(lane == k, sc, p1)
                    p2 = jnp.where(lane == k - 16, sc, p2)
                    return (p1, p2)

                neg = jnp.full((16,), -3.0e38, jnp.float32)
                p1, p2 = lax.fori_loop(0, NEIGH, k_score, (neg, neg))

                m = _bmax(jnp.maximum(p1, p2), lane)
                e1 = jnp.exp(p1 - m)
                e2 = jnp.exp(p2 - m)
                inv = 1.0 / _bsum(e1 + e2, lane)
                sc_v[pl.ds(0, 16)] = e1 * inv
                sc_v[pl.ds(16, 16)] = e2 * inv

                def k_out(k, out):
                    f = f0 + k
                    p = _gather16(sc_v[pl.ds(k, 16)], lane0)
                    return tuple(out[c] + p * t_v[f, pl.ds(c * 16, 16)]
                                 for c in range(C))

                out = lax.fori_loop(0, NEIGH, k_out,
                                    tuple(zero16 for _ in range(C)))
                for c in range(C):
                    out_v[i, pl.ds(c * 16, 16)] = out[c]
                return 0

            lax.fori_loop(0, BATCH, row_body, 0)
            pltpu.sync_copy(out_v, out_hbm.at[pl.ds(s, BATCH)])
            return 0

        lax.fori_loop(0, nb, batch_body, 0)

    return agg


_agg_news = _make_agg(NEWS_PAD)
_agg_ent = _make_agg(ENT_PAD)


def _aggregate(head_emb, entity_emb, relation_emb, nbr_entities, nbr_relations,
               agg_fn, n, n_pad):
    pad = n_pad - n
    head_p = jnp.concatenate(
        [head_emb, jnp.zeros((pad, D), jnp.float32)], axis=0)
    idx_p = jnp.concatenate(
        [nbr_entities, jnp.zeros((pad, NEIGH), jnp.int32)], axis=0)
    rel_p = jnp.concatenate(
        [nbr_relations, jnp.zeros((pad, NEIGH), jnp.int32)], axis=0)
    idx_r = idx_p.reshape(n_pad // BATCH, IDX_ROWS, IDX_COLS)
    rel_r = rel_p.reshape(n_pad // BATCH, IDX_ROWS, IDX_COLS)
    out = agg_fn(head_p, idx_r, rel_r, entity_emb, relation_emb)
    return out[:n]


def _user_stage_kernel(user_ref, inter_ref, agg_ref, out_ref):
    agg = agg_ref[...]
    ua = jnp.dot(inter_ref[...], agg, preferred_element_type=jnp.float32)
    logits = jnp.dot(user_ref[...], agg.T, preferred_element_type=jnp.float32)
    m = jnp.max(logits, axis=-1, keepdims=True)
    e = jnp.exp(logits - m)
    s = e / jnp.sum(e, axis=-1, keepdims=True)
    sa = jnp.dot(s, agg, preferred_element_type=jnp.float32)
    out_ref[...] = ua + sa * ua


def _user_stage(user_emb, interact_mat, news_agg):
    grid = (N_USERS // USER_BLK,)
    return pl.pallas_call(
        _user_stage_kernel,
        grid=grid,
        in_specs=[
            pl.BlockSpec((USER_BLK, D), lambda i: (i, 0)),
            pl.BlockSpec((USER_BLK, N_NEWS), lambda i: (i, 0)),
            pl.BlockSpec((N_NEWS, D), lambda i: (0, 0)),
        ],
        out_specs=pl.BlockSpec((USER_BLK, D), lambda i: (i, 0)),
        out_shape=jax.ShapeDtypeStruct((N_USERS, D), jnp.float32),
    )(user_emb, interact_mat, news_agg)


def kernel(user_emb, news_embeding, entity_emb, relation_emb, interact_mat,
           news_entities, news_relations, neigh_entities, neigh_relations):
    news_agg = _aggregate(news_embeding, entity_emb, relation_emb,
                          news_entities, news_relations,
                          _agg_news, N_NEWS, NEWS_PAD)
    entity_agg = _aggregate(entity_emb, entity_emb, relation_emb,
                            neigh_entities, neigh_relations,
                            _agg_ent, N_ENTITY, ENT_PAD)
    user_agg = _user_stage(user_emb, interact_mat, news_agg)
    return (news_agg, entity_agg, user_agg)


# speedup vs baseline: 4.2820x; 4.2820x over previous
import functools

import jax
import jax.numpy as jnp
from jax import lax
from jax.experimental import pallas as pl
from jax.experimental.pallas import tpu as pltpu
from jax.experimental.pallas import tpu_sc as plsc

N_USERS = 4096
N_NEWS = 10000
N_ENTITY = 30000
N_REL = 40
D = 128
NEIGH = 20

C = D // 16
BATCH = 16
IDX_PER_BATCH = BATCH * NEIGH
IDX_ROWS = 4
IDX_COLS = IDX_PER_BATCH // IDX_ROWS
NWORKERS = 32

NEWS_PAD = 10240
ENT_PAD = 30720

USER_BLK = 256

_GDN = lax.GatherDimensionNumbers(
    offset_dims=(), collapsed_slice_dims=(0,), start_index_map=(0,))


def _gather16(v, idx):
    return lax.gather(v, idx[:, None], dimension_numbers=_GDN,
                      slice_sizes=(1,),
                      mode=lax.GatherScatterMode.PROMISE_IN_BOUNDS)


def _bsum(v, lane):
    for s in (1, 2, 4, 8):
        v = v + _gather16(v, lax.bitwise_xor(lane, s))
    return v


def _bmax(v, lane):
    for s in (1, 2, 4, 8):
        v = jnp.maximum(v, _gather16(v, lax.bitwise_xor(lane, s)))
    return v


def _bf16_round(x):
    i = lax.bitcast_convert_type(x, jnp.int32)
    r = i + 0x7FFF + lax.bitwise_and(lax.shift_right_logical(i, 16), 1)
    return lax.bitcast_convert_type(lax.bitwise_and(r, -65536), jnp.float32)


def _round_pair(a, b):
    return _bf16_round(a), _bf16_round(b)


def _round8(xs):
    return [_bf16_round(x) for x in xs]


def _sqrt16(x):
    xs = jnp.maximum(x, 1e-30)
    i = lax.bitcast_convert_type(xs, jnp.int32)
    y = lax.bitcast_convert_type(0x5F3759DF - lax.shift_right_arithmetic(i, 1),
                                 jnp.float32)
    for _ in range(3):
        y = y * (1.5 - 0.5 * xs * y * y)
    return xs * y


def _make_agg(n_pad, news_mode):
    nb = n_pad // (NWORKERS * BATCH)
    mesh = plsc.VectorSubcoreMesh(core_axis_name="c", subcore_axis_name="s")

    scratch = [
        pltpu.VMEM((IDX_ROWS, IDX_COLS), jnp.int32),
        pltpu.VMEM((BATCH, D), jnp.float32),
        pltpu.VMEM((IDX_PER_BATCH, D), jnp.float32),
        pltpu.VMEM((48,), jnp.float32),
        pltpu.VMEM((BATCH, D), jnp.float32),
        pltpu.SemaphoreType.DMA,
    ]
    if news_mode:
        scratch.insert(5, pltpu.VMEM((D,), jnp.float32))
        scratch.insert(6, pltpu.VMEM((D,), jnp.float32))
    else:
        scratch.insert(1, pltpu.VMEM((IDX_ROWS, IDX_COLS), jnp.int32))
        scratch.insert(4, pltpu.VMEM((IDX_PER_BATCH, D), jnp.float32))

    def body(refs):
        if news_mode:
            (head_hbm, idx_hbm, ent_hbm, relemb_hbm, out_hbm,
             idx_v, h_v, t_v, sc_v, out_v, rel0_v, sq_v, sem) = refs
        else:
            (head_hbm, idx_hbm, relid_hbm, ent_hbm, relemb_hbm, out_hbm,
             idx_v, relid_v, h_v, t_v, r_v, sc_v, out_v, sem) = refs
        wid = lax.axis_index("s") * 2 + lax.axis_index("c")
        lane = lax.broadcasted_iota(jnp.int32, (16,), 0)
        lane0 = jnp.zeros((16,), jnp.int32)
        zero16 = jnp.zeros((16,), jnp.float32)
        neg16 = jnp.full((16,), -3.0e38, jnp.float32)

        if news_mode:
            pltpu.sync_copy(relemb_hbm.at[pl.ds(0, D)], rel0_v)
            for c in range(C):
                v = rel0_v[pl.ds(c * 16, 16)]
                vv = v * v
                acc = vv
                for _ in range(NEIGH - 1):
                    acc = acc + vv
                sq_v[pl.ds(c * 16, 16)] = _sqrt16(acc)

        def batch_body(b, _):
            gb = wid * nb + b
            s = gb * BATCH

            pltpu.sync_copy(idx_hbm.at[gb], idx_v)
            if not news_mode:
                pltpu.sync_copy(relid_hbm.at[gb], relid_v)
            pltpu.sync_copy(head_hbm.at[pl.ds(s, BATCH)], h_v)
            copies = [
                pltpu.async_copy(ent_hbm.at[idx_v.at[j]],
                                 t_v.at[pl.ds(j * IDX_COLS, IDX_COLS)], sem)
                for j in range(IDX_ROWS)
            ]
            if not news_mode:
                copies += [
                    pltpu.async_copy(relemb_hbm.at[relid_v.at[j]],
                                     r_v.at[pl.ds(j * IDX_COLS, IDX_COLS)],
                                     sem)
                    for j in range(IDX_ROWS)
                ]
            for cp in copies:
                cp.wait()

            def row_body(i, _):
                f0 = i * NEIGH
                h_c = [h_v[i, pl.ds(c * 16, 16)] for c in range(C)]

                if news_mode:
                    sq_c = [sq_v[pl.ds(c * 16, 16)] for c in range(C)]
                    r0_c = [rel0_v[pl.ds(c * 16, 16)] for c in range(C)]
                    w_c = _round8([jnp.abs(h_c[c]) * sq_c[c]
                                   for c in range(C)])
                else:
                    s2 = [zero16] * C
                    for k in range(NEIGH):
                        f = f0 + k
                        for c in range(C):
                            rl = r_v[f, pl.ds(c * 16, 16)]
                            t = t_v[f, pl.ds(c * 16, 16)]
                            s2[c] = s2[c] + rl * rl
                            r_v[f, pl.ds(c * 16, 16)] = _bf16_round(t * rl)
                    w_c = _round8([jnp.abs(h_c[c]) * _sqrt16(s2[c])
                                   for c in range(C)])

                p1 = neg16
                p2 = neg16
                for k in range(NEIGH):
                    f = f0 + k
                    acc = zero16
                    for c in range(C):
                        if news_mode:
                            u = _bf16_round(t_v[f, pl.ds(c * 16, 16)]
                                            * r0_c[c])
                        else:
                            u = r_v[f, pl.ds(c * 16, 16)]
                        acc = acc + u * w_c[c]
                    sc = _bsum(acc, lane)
                    sc = sc * sc
                    if k < 16:
                        p1 = jnp.where(lane == k, sc, p1)
                    else:
                        p2 = jnp.where(lane == k - 16, sc, p2)

                m = _bmax(jnp.maximum(p1, p2), lane)
                e1 = jnp.exp(p1 - m)
                e2 = jnp.exp(p2 - m)
                inv = 1.0 / _bsum(e1 + e2, lane)
                sc_v[pl.ds(0, 16)] = e1 * inv
                sc_v[pl.ds(16, 16)] = e2 * inv

                out = [zero16] * C
                for k in range(NEIGH):
                    f = f0 + k
                    p = _gather16(sc_v[pl.ds(k, 16)], lane0)
                    for c in range(C):
                        out[c] = out[c] + p * t_v[f, pl.ds(c * 16, 16)]
                for c in range(C):
                    out_v[i, pl.ds(c * 16, 16)] = out[c]
                return 0

            lax.fori_loop(0, BATCH, row_body, 0)
            pltpu.sync_copy(out_v, out_hbm.at[pl.ds(s, BATCH)])
            return 0

        lax.fori_loop(0, nb, batch_body, 0)

    @functools.partial(
        pl.kernel,
        out_type=jax.ShapeDtypeStruct((n_pad, D), jnp.float32),
        mesh=mesh,
        scratch_types=scratch,
    )
    def agg(*refs):
        body(refs)

    return agg


_agg_news = _make_agg(NEWS_PAD, True)
_agg_ent = _make_agg(ENT_PAD, False)


def _aggregate(head_emb, entity_emb, relation_emb, nbr_entities, nbr_relations,
               agg_fn, n, n_pad, news_mode):
    pad = n_pad - n
    head_p = jnp.concatenate(
        [head_emb, jnp.zeros((pad, D), jnp.float32)], axis=0)
    idx_p = jnp.concatenate(
        [nbr_entities, jnp.zeros((pad, NEIGH), jnp.int32)], axis=0)
    idx_r = idx_p.reshape(n_pad // BATCH, IDX_ROWS, IDX_COLS)
    if news_mode:
        out = agg_fn(head_p, idx_r, entity_emb, relation_emb.reshape(-1))
    else:
        rel_p = jnp.concatenate(
            [nbr_relations, jnp.zeros((pad, NEIGH), jnp.int32)], axis=0)
        rel_r = rel_p.reshape(n_pad // BATCH, IDX_ROWS, IDX_COLS)
        out = agg_fn(head_p, idx_r, rel_r, entity_emb, relation_emb)
    return out[:n]


def _user_stage_kernel(user_ref, inter_ref, agg_ref, out_ref):
    agg = agg_ref[...]
    ua = jnp.dot(inter_ref[...], agg, preferred_element_type=jnp.float32)
    logits = jnp.dot(user_ref[...], agg.T, preferred_element_type=jnp.float32)
    m = jnp.max(logits, axis=-1, keepdims=True)
    e = jnp.exp(logits - m)
    s = e / jnp.sum(e, axis=-1, keepdims=True)
    sa = jnp.dot(s, agg, preferred_element_type=jnp.float32)
    out_ref[...] = ua + sa * ua


def _user_stage(user_emb, interact_mat, news_agg):
    grid = (N_USERS // USER_BLK,)
    return pl.pallas_call(
        _user_stage_kernel,
        grid=grid,
        in_specs=[
            pl.BlockSpec((USER_BLK, D), lambda i: (i, 0)),
            pl.BlockSpec((USER_BLK, N_NEWS), lambda i: (i, 0)),
            pl.BlockSpec((N_NEWS, D), lambda i: (0, 0)),
        ],
        out_specs=pl.BlockSpec((USER_BLK, D), lambda i: (i, 0)),
        out_shape=jax.ShapeDtypeStruct((N_USERS, D), jnp.float32),
    )(user_emb, interact_mat, news_agg)


def kernel(user_emb, news_embeding, entity_emb, relation_emb, interact_mat,
           news_entities, news_relations, neigh_entities, neigh_relations):
    news_agg = _aggregate(news_embeding, entity_emb, relation_emb,
                          news_entities, news_relations,
                          _agg_news, N_NEWS, NEWS_PAD, True)
    entity_agg = _aggregate(entity_emb, entity_emb, relation_emb,
                            neigh_entities, neigh_relations,
                            _agg_ent, N_ENTITY, ENT_PAD, False)
    user_agg = _user_stage(user_emb, interact_mat, news_agg)
    return (news_agg, entity_agg, user_agg)
